# baseline (speedup 1.0000x reference)
import jax
import jax.numpy as jnp
from jax.experimental import pallas as pl

_K = 1024
_D = 256
_H = 24
_W = 24
_P = _H * _W
_BPS = 2


def _vq_body(z_ref, cb_ref, out_ref):
    cb = cb_ref[...]
    c_sqr = jnp.sum(cb * cb, axis=1, keepdims=True)
    iota = jax.lax.broadcasted_iota(jnp.int32, (_K, 1), 0).astype(jnp.float32)
    for i in range(_BPS):
        zs = -2.0 * z_ref[i].reshape(_D, _P)
        mm2 = jnp.dot(cb, zs, preferred_element_type=jnp.float32)
        x_sqr = 0.25 * jnp.sum(zs * zs, axis=0, keepdims=True)
        dist = (c_sqr + x_sqr) + mm2
        m = jnp.min(dist, axis=0, keepdims=True)
        cand = jnp.where(dist == m, iota, float(_K))
        out_ref[0, i, :] = jnp.min(cand, axis=0).astype(jnp.int32)


@jax.jit
def kernel(z_e_x, codebook):
    b, c, h, w = z_e_x.shape
    n_blocks = b // _BPS
    out = pl.pallas_call(
        _vq_body,
        grid=(n_blocks,),
        in_specs=[
            pl.BlockSpec((_BPS, _D, _H, _W), lambda i: (i, 0, 0, 0)),
            pl.BlockSpec((_K, _D), lambda i: (0, 0)),
        ],
        out_specs=pl.BlockSpec((1, _BPS, _P), lambda i: (i, 0, 0)),
        out_shape=jax.ShapeDtypeStruct((n_blocks, _BPS, _P), jnp.int32),
    )(z_e_x, codebook)
    return out.reshape(b, h, w)


# speedup vs baseline: 3.1689x; 3.1689x over previous
import jax
import jax.numpy as jnp
from jax.experimental import pallas as pl

_K = 1024
_D = 256
_RB = 1152


def _vq_body(x_ref, cb_ref, c2_ref, out_ref):
    xs = -2.0 * x_ref[...]
    cb = cb_ref[...]
    mm2 = jax.lax.dot_general(
        xs, cb, (((1,), (1,)), ((), ())),
        preferred_element_type=jnp.float32,
    )
    x_sqr = 0.25 * jnp.sum(xs * xs, axis=1, keepdims=True)
    dist = (c2_ref[...] + x_sqr) + mm2
    m = jnp.min(dist, axis=1, keepdims=True)
    iota = jax.lax.broadcasted_iota(jnp.int32, (1, _K), 1).astype(jnp.float32)
    cand = jnp.where(dist == m, iota, float(_K))
    idx = jnp.min(cand, axis=1, keepdims=True)
    out_ref[...] = idx.astype(jnp.int32)


@jax.jit
def kernel(z_e_x, codebook):
    b, c, h, w = z_e_x.shape
    n = b * h * w
    x = jnp.transpose(z_e_x, (0, 2, 3, 1)).reshape(n, c)
    c_sqr = jnp.sum(codebook ** 2, axis=1)[None, :]
    out = pl.pallas_call(
        _vq_body,
        grid=(n // _RB,),
        in_specs=[
            pl.BlockSpec((_RB, _D), lambda i: (i, 0)),
            pl.BlockSpec((_K, _D), lambda i: (0, 0)),
            pl.BlockSpec((1, _K), lambda i: (0, 0)),
        ],
        out_specs=pl.BlockSpec((_RB, 1), lambda i: (i, 0)),
        out_shape=jax.ShapeDtypeStruct((n, 1), jnp.int32),
    )(x, codebook, c_sqr)
    return out.reshape(b, h, w)


# speedup vs baseline: 3.6272x; 1.1446x over previous
import jax
import jax.numpy as jnp
from jax.experimental import pallas as pl

_K = 1024
_D = 256
_RB = 2304


def _vq_body(x_ref, cb_ref, c2_ref, out_ref):
    xs = -2.0 * x_ref[...]
    cb = cb_ref[...]
    mm2 = jax.lax.dot_general(
        xs, cb, (((1,), (1,)), ((), ())),
        preferred_element_type=jnp.float32,
    )
    x_sqr = 0.25 * jnp.sum(xs * xs, axis=1, keepdims=True)
    dist = (c2_ref[...] + x_sqr) + mm2
    m = jnp.min(dist, axis=1, keepdims=True)
    iota = jax.lax.broadcasted_iota(jnp.int32, (1, _K), 1).astype(jnp.float32)
    cand = jnp.where(dist == m, iota, float(_K))
    idx = jnp.min(cand, axis=1, keepdims=True)
    out_ref[...] = idx.astype(jnp.int32)


@jax.jit
def kernel(z_e_x, codebook):
    b, c, h, w = z_e_x.shape
    n = b * h * w
    x = jnp.transpose(z_e_x, (0, 2, 3, 1)).reshape(n, c)
    c_sqr = jnp.sum(codebook ** 2, axis=1)[None, :]
    out = pl.pallas_call(
        _vq_body,
        grid=(n // _RB,),
        in_specs=[
            pl.BlockSpec((_RB, _D), lambda i: (i, 0)),
            pl.BlockSpec((_K, _D), lambda i: (0, 0)),
            pl.BlockSpec((1, _K), lambda i: (0, 0)),
        ],
        out_specs=pl.BlockSpec((_RB, 1), lambda i: (i, 0)),
        out_shape=jax.ShapeDtypeStruct((n, 1), jnp.int32),
    )(x, codebook, c_sqr)
    return out.reshape(b, h, w)


# speedup vs baseline: 3.7371x; 1.0303x over previous
import jax
import jax.numpy as jnp
from jax.experimental import pallas as pl

_K = 1024
_D = 256
_RB = 4608


def _vq_body(x_ref, cb_ref, c2_ref, out_ref):
    xs = -2.0 * x_ref[...]
    cb = cb_ref[...]
    mm2 = jax.lax.dot_general(
        xs, cb, (((1,), (1,)), ((), ())),
        preferred_element_type=jnp.float32,
    )
    x_sqr = 0.25 * jnp.sum(xs * xs, axis=1, keepdims=True)
    dist = (c2_ref[...] + x_sqr) + mm2
    m = jnp.min(dist, axis=1, keepdims=True)
    iota = jax.lax.broadcasted_iota(jnp.int32, (1, _K), 1).astype(jnp.float32)
    cand = jnp.where(dist == m, iota, float(_K))
    idx = jnp.min(cand, axis=1, keepdims=True)
    out_ref[...] = idx.astype(jnp.int32)


@jax.jit
def kernel(z_e_x, codebook):
    b, c, h, w = z_e_x.shape
    n = b * h * w
    x = jnp.transpose(z_e_x, (0, 2, 3, 1)).reshape(n, c)
    c_sqr = jnp.sum(codebook ** 2, axis=1)[None, :]
    out = pl.pallas_call(
        _vq_body,
        grid=(n // _RB,),
        in_specs=[
            pl.BlockSpec((_RB, _D), lambda i: (i, 0)),
            pl.BlockSpec((_K, _D), lambda i: (0, 0)),
            pl.BlockSpec((1, _K), lambda i: (0, 0)),
        ],
        out_specs=pl.BlockSpec((_RB, 1), lambda i: (i, 0)),
        out_shape=jax.ShapeDtypeStruct((n, 1), jnp.int32),
    )(x, codebook, c_sqr)
    return out.reshape(b, h, w)
